# baseline (speedup 1.0000x reference)
import functools

import jax
import jax.numpy as jnp
from jax import lax
from jax.experimental import pallas as pl
from jax.experimental.pallas import tpu as pltpu
from jax.experimental.pallas import tpu_sc as plsc

N_NODES = 10000
D = 256
N_TILES = 32
ROWS = 312
ROWS_LAST = N_NODES - (N_TILES - 1) * ROWS
TRASH = 328
ACC_ROWS = 336
CHUNK_E = 2048
N_CHUNKS = 80
E_PAD = CHUNK_E * N_CHUNKS
BLK = 32
L = 16
UNROLL = 8


def _linear_body(x_ref, w_ref, b_ref, h_ref):
    h_ref[...] = (
        jnp.dot(x_ref[...], w_ref[...], preferred_element_type=jnp.float32)
        + b_ref[...]
    )


def _linear(x, W, b):
    return pl.pallas_call(
        _linear_body,
        grid=(10,),
        in_specs=[
            pl.BlockSpec((1000, D), lambda i: (i, 0)),
            pl.BlockSpec((D, D), lambda i: (0, 0)),
            pl.BlockSpec((1, D), lambda i: (0, 0)),
        ],
        out_specs=pl.BlockSpec((1000, D), lambda i: (i, 0)),
        out_shape=jax.ShapeDtypeStruct((N_NODES, D), jnp.float32),
    )(x, W, b.reshape(1, D))


def _sc_aggregate(h, src, dst):
    mesh = plsc.VectorSubcoreMesh(core_axis_name="c", subcore_axis_name="s")

    @functools.partial(
        pl.kernel,
        mesh=mesh,
        out_type=jax.ShapeDtypeStruct((N_NODES, D), jnp.float32),
        compiler_params=pltpu.CompilerParams(needs_layout_passes=False),
        scratch_types=[
            pltpu.VMEM((CHUNK_E,), jnp.int32),
            pltpu.VMEM((CHUNK_E,), jnp.int32),
            pltpu.VMEM((CHUNK_E,), jnp.int32),
            pltpu.VMEM((CHUNK_E,), jnp.int32),
            pltpu.VMEM((CHUNK_E + BLK,), jnp.int32),
            pltpu.VMEM((CHUNK_E + BLK,), jnp.int32),
            pltpu.VMEM((BLK, D), jnp.float32),
            pltpu.VMEM((BLK, D), jnp.float32),
            pltpu.VMEM((ACC_ROWS, D), jnp.float32),
            pltpu.SemaphoreType.DMA,
            pltpu.SemaphoreType.DMA,
            pltpu.SemaphoreType.DMA,
            pltpu.SemaphoreType.DMA,
        ],
    )
    def agg(h_hbm, src_hbm, dst_hbm, out_hbm,
            srcb0, dstb0, srcb1, dstb1, psrc, pldst, rows0, rows1, accum,
            sem0, sem1, stg0, stg1):
        cid = lax.axis_index("c")
        sid = lax.axis_index("s")
        wid = sid * 2 + cid
        row0 = wid * ROWS
        last = wid == N_TILES - 1
        nrows = jnp.where(last, ROWS_LAST, ROWS).astype(jnp.int32)
        lanes = lax.iota(jnp.int32, L)

        @pl.when(jnp.logical_not(last))
        def _():
            pltpu.sync_copy(h_hbm.at[pl.ds(row0, ROWS)], accum.at[pl.ds(0, ROWS)])

        @pl.when(last)
        def _():
            pltpu.sync_copy(
                h_hbm.at[pl.ds((N_TILES - 1) * ROWS, ROWS_LAST)],
                accum.at[pl.ds(0, ROWS_LAST)],
            )

        def stage_start(ch, sb, db, sgm):
            e0 = ch * CHUNK_E
            pltpu.make_async_copy(src_hbm.at[pl.ds(e0, CHUNK_E)], sb, sgm).start()
            pltpu.make_async_copy(dst_hbm.at[pl.ds(e0, CHUNK_E)], db, sgm).start()

        def stage_wait(sb, db, sgm):
            pltpu.make_async_copy(src_hbm.at[pl.ds(0, CHUNK_E)], sb, sgm).wait()
            pltpu.make_async_copy(dst_hbm.at[pl.ds(0, CHUNK_E)], db, sgm).wait()

        def gather_start(blkv, rows_b, sem_b):
            pltpu.make_async_copy(
                h_hbm.at[psrc.at[pl.ds(blkv * BLK, BLK)]], rows_b, sem_b
            ).start()

        def gather_wait(rows_b, sem_b):
            pltpu.make_async_copy(
                h_hbm.at[psrc.at[pl.ds(0, BLK)]], rows_b, sem_b
            ).wait()

        def add_block(blk, rows_b):
            def group_body(g, c2):
                ridx = pldst[pl.ds(blk * BLK + g * L, L)]
                for l in range(L):
                    r = jnp.sum(jnp.where(lanes == l, ridx, 0))
                    for cb in range(D // L):
                        vals = rows_b[g * L + l, pl.ds(cb * L, L)]
                        plsc.addupdate(accum.at[r, pl.ds(cb * L, L)], vals)
                return c2

            lax.fori_loop(0, BLK // L, group_body, 0)

        def chunk_work(srcb, dstb):
            def scan_group(g0, curv):
                for u in range(UNROLL):
                    g = g0 * UNROLL + u
                    d = dstb[pl.ds(g * L, L)]
                    s = srcb[pl.ds(g * L, L)]
                    adj = d - row0
                    m = (adj >= 0) & (adj < nrows)
                    mi = m.astype(jnp.int32)
                    inc = plsc.cumsum(mi)
                    pos = curv + inc - 1
                    plsc.store_scatter(pldst, [pos], adj, mask=m)
                    plsc.store_scatter(psrc, [pos], s, mask=m)
                    curv = curv + plsc.all_reduce_population_count(m)
                return curv

            curv = lax.fori_loop(
                0, CHUNK_E // L // UNROLL, scan_group,
                jnp.zeros((L,), jnp.int32),
            )
            cur = jnp.max(curv)

            nblk = (cur + BLK - 1) // BLK
            pad_end = nblk * BLK
            for t in range(BLK // L):
                off = t * L + lanes
                ppos = cur + off
                pm = ppos < pad_end
                plsc.store_scatter(
                    pldst, [ppos], jnp.full((L,), TRASH, jnp.int32), mask=pm
                )
                plsc.store_scatter(
                    psrc, [ppos], jnp.broadcast_to(row0, (L,)).astype(jnp.int32),
                    mask=pm,
                )

            @pl.when(nblk > 0)
            def _():
                gather_start(0, rows0, sem0)

            def pair_body(p, c2):
                b0 = 2 * p

                @pl.when(b0 + 1 < nblk)
                def _():
                    gather_start(b0 + 1, rows1, sem1)

                gather_wait(rows0, sem0)
                add_block(b0, rows0)

                @pl.when(b0 + 2 < nblk)
                def _():
                    gather_start(b0 + 2, rows0, sem0)

                @pl.when(b0 + 1 < nblk)
                def _():
                    gather_wait(rows1, sem1)
                    add_block(b0 + 1, rows1)

                return c2

            lax.fori_loop(0, (nblk + 1) // 2, pair_body, 0)

        stage_start(0, srcb0, dstb0, stg0)

        def chunk_pair(p, carry):
            ch0 = 2 * p
            stage_start(ch0 + 1, srcb1, dstb1, stg1)
            stage_wait(srcb0, dstb0, stg0)
            chunk_work(srcb0, dstb0)

            @pl.when(ch0 + 2 < N_CHUNKS)
            def _():
                stage_start(ch0 + 2, srcb0, dstb0, stg0)

            stage_wait(srcb1, dstb1, stg1)
            chunk_work(srcb1, dstb1)
            return carry

        lax.fori_loop(0, N_CHUNKS // 2, chunk_pair, 0)

        @pl.when(jnp.logical_not(last))
        def _():
            pltpu.sync_copy(
                accum.at[pl.ds(0, ROWS)], out_hbm.at[pl.ds(row0, ROWS)]
            )

        @pl.when(last)
        def _():
            pltpu.sync_copy(
                accum.at[pl.ds(0, ROWS_LAST)],
                out_hbm.at[pl.ds((N_TILES - 1) * ROWS, ROWS_LAST)],
            )

    return agg(h, src, dst)


def kernel(x, edge_index, W, b):
    e = edge_index.astype(jnp.int32)
    pad = E_PAD - e.shape[1]
    src = jnp.concatenate([e[0], jnp.zeros((pad,), jnp.int32)])
    dst = jnp.concatenate([e[1], jnp.full((pad,), -1, jnp.int32)])
    h = _linear(x, W, b)
    return _sc_aggregate(h, src, dst)


# speedup vs baseline: 1.2180x; 1.2180x over previous
import functools

import jax
import jax.numpy as jnp
from jax import lax
from jax.experimental import pallas as pl
from jax.experimental.pallas import tpu as pltpu
from jax.experimental.pallas import tpu_sc as plsc

N_NODES = 10000
D = 256
N_TILES = 32
ROWS = 312
ROWS_LAST = N_NODES - (N_TILES - 1) * ROWS
TRASH = 328
ACC_ROWS = 336
CHUNK_E = 4096
N_CHUNKS = 40
E_PAD = CHUNK_E * N_CHUNKS
BLK = 32
L = 16
UNROLL = 4


def _linear_body(x_ref, w_ref, b_ref, h_ref):
    h_ref[...] = (
        jnp.dot(x_ref[...], w_ref[...], preferred_element_type=jnp.float32)
        + b_ref[...]
    )


def _linear(x, W, b):
    return pl.pallas_call(
        _linear_body,
        grid=(10,),
        in_specs=[
            pl.BlockSpec((1000, D), lambda i: (i, 0)),
            pl.BlockSpec((D, D), lambda i: (0, 0)),
            pl.BlockSpec((1, D), lambda i: (0, 0)),
        ],
        out_specs=pl.BlockSpec((1000, D), lambda i: (i, 0)),
        out_shape=jax.ShapeDtypeStruct((N_NODES, D), jnp.float32),
    )(x, W, b.reshape(1, D))


def _sc_aggregate(h, src, dst):
    mesh = plsc.VectorSubcoreMesh(core_axis_name="c", subcore_axis_name="s")

    @functools.partial(
        pl.kernel,
        mesh=mesh,
        out_type=jax.ShapeDtypeStruct((N_NODES, D), jnp.float32),
        compiler_params=pltpu.CompilerParams(needs_layout_passes=False),
        scratch_types=[
            pltpu.VMEM((CHUNK_E,), jnp.int32),
            pltpu.VMEM((CHUNK_E,), jnp.int32),
            pltpu.VMEM((CHUNK_E,), jnp.int32),
            pltpu.VMEM((CHUNK_E,), jnp.int32),
            pltpu.VMEM((CHUNK_E + BLK,), jnp.int32),
            pltpu.VMEM((CHUNK_E + BLK,), jnp.int32),
            pltpu.VMEM((BLK, D), jnp.float32),
            pltpu.VMEM((BLK, D), jnp.float32),
            pltpu.VMEM((ACC_ROWS, D), jnp.float32),
            pltpu.SemaphoreType.DMA,
            pltpu.SemaphoreType.DMA,
            pltpu.SemaphoreType.DMA,
            pltpu.SemaphoreType.DMA,
        ],
    )
    def agg(h_hbm, src_hbm, dst_hbm, out_hbm,
            srcb0, dstb0, srcb1, dstb1, psrc, pldst, rows0, rows1, accum,
            sem0, sem1, stg0, stg1):
        cid = lax.axis_index("c")
        sid = lax.axis_index("s")
        wid = sid * 2 + cid
        row0 = wid * ROWS
        last = wid == N_TILES - 1
        nrows = jnp.where(last, ROWS_LAST, ROWS).astype(jnp.int32)
        lanes = lax.iota(jnp.int32, L)

        @pl.when(jnp.logical_not(last))
        def _():
            pltpu.sync_copy(h_hbm.at[pl.ds(row0, ROWS)], accum.at[pl.ds(0, ROWS)])

        @pl.when(last)
        def _():
            pltpu.sync_copy(
                h_hbm.at[pl.ds((N_TILES - 1) * ROWS, ROWS_LAST)],
                accum.at[pl.ds(0, ROWS_LAST)],
            )

        def stage_start(ch, sb, db, sgm):
            e0 = ch * CHUNK_E
            pltpu.make_async_copy(src_hbm.at[pl.ds(e0, CHUNK_E)], sb, sgm).start()
            pltpu.make_async_copy(dst_hbm.at[pl.ds(e0, CHUNK_E)], db, sgm).start()

        def stage_wait(sb, db, sgm):
            pltpu.make_async_copy(src_hbm.at[pl.ds(0, CHUNK_E)], sb, sgm).wait()
            pltpu.make_async_copy(dst_hbm.at[pl.ds(0, CHUNK_E)], db, sgm).wait()

        def gather_start(blkv, rows_b, sem_b):
            pltpu.make_async_copy(
                h_hbm.at[psrc.at[pl.ds(blkv * BLK, BLK)]], rows_b, sem_b
            ).start()

        def gather_wait(rows_b, sem_b):
            pltpu.make_async_copy(
                h_hbm.at[psrc.at[pl.ds(0, BLK)]], rows_b, sem_b
            ).wait()

        def add_block(blk, rows_b):
            def group_body(g, c2):
                ridx = pldst[pl.ds(blk * BLK + g * L, L)]
                for l in range(L):
                    r = jnp.sum(jnp.where(lanes == l, ridx, 0))
                    for cb in range(D // L):
                        vals = rows_b[g * L + l, pl.ds(cb * L, L)]
                        plsc.addupdate(accum.at[r, pl.ds(cb * L, L)], vals)
                return c2

            lax.fori_loop(0, BLK // L, group_body, 0)

        def chunk_work(srcb, dstb):
            def scan_group(g0, curv):
                for u in range(UNROLL):
                    g = g0 * UNROLL + u
                    d = dstb[pl.ds(g * L, L)]
                    s = srcb[pl.ds(g * L, L)]
                    adj = d - row0
                    m = (adj >= 0) & (adj < nrows)
                    mi = m.astype(jnp.int32)
                    inc = plsc.cumsum(mi)
                    pos = curv + inc - 1
                    plsc.store_scatter(pldst, [pos], adj, mask=m)
                    plsc.store_scatter(psrc, [pos], s, mask=m)
                    curv = curv + plsc.all_reduce_population_count(m)
                return curv

            curv = lax.fori_loop(
                0, CHUNK_E // L // UNROLL, scan_group,
                jnp.zeros((L,), jnp.int32),
            )
            cur = jnp.max(curv)

            nblk = (cur + BLK - 1) // BLK
            pad_end = nblk * BLK
            for t in range(BLK // L):
                off = t * L + lanes
                ppos = cur + off
                pm = ppos < pad_end
                plsc.store_scatter(
                    pldst, [ppos], jnp.full((L,), TRASH, jnp.int32), mask=pm
                )
                plsc.store_scatter(
                    psrc, [ppos], jnp.broadcast_to(row0, (L,)).astype(jnp.int32),
                    mask=pm,
                )

            @pl.when(nblk > 0)
            def _():
                gather_start(0, rows0, sem0)

            def pair_body(p, c2):
                b0 = 2 * p

                @pl.when(b0 + 1 < nblk)
                def _():
                    gather_start(b0 + 1, rows1, sem1)

                gather_wait(rows0, sem0)
                add_block(b0, rows0)

                @pl.when(b0 + 2 < nblk)
                def _():
                    gather_start(b0 + 2, rows0, sem0)

                @pl.when(b0 + 1 < nblk)
                def _():
                    gather_wait(rows1, sem1)
                    add_block(b0 + 1, rows1)

                return c2

            lax.fori_loop(0, (nblk + 1) // 2, pair_body, 0)

        stage_start(0, srcb0, dstb0, stg0)

        def chunk_pair(p, carry):
            ch0 = 2 * p
            stage_start(ch0 + 1, srcb1, dstb1, stg1)
            stage_wait(srcb0, dstb0, stg0)
            chunk_work(srcb0, dstb0)

            @pl.when(ch0 + 2 < N_CHUNKS)
            def _():
                stage_start(ch0 + 2, srcb0, dstb0, stg0)

            stage_wait(srcb1, dstb1, stg1)
            chunk_work(srcb1, dstb1)
            return carry

        lax.fori_loop(0, N_CHUNKS // 2, chunk_pair, 0)

        @pl.when(jnp.logical_not(last))
        def _():
            pltpu.sync_copy(
                accum.at[pl.ds(0, ROWS)], out_hbm.at[pl.ds(row0, ROWS)]
            )

        @pl.when(last)
        def _():
            pltpu.sync_copy(
                accum.at[pl.ds(0, ROWS_LAST)],
                out_hbm.at[pl.ds((N_TILES - 1) * ROWS, ROWS_LAST)],
            )

    return agg(h, src, dst)


def kernel(x, edge_index, W, b):
    e = edge_index.astype(jnp.int32)
    pad = E_PAD - e.shape[1]
    src = jnp.concatenate([e[0], jnp.zeros((pad,), jnp.int32)])
    dst = jnp.concatenate([e[1], jnp.full((pad,), -1, jnp.int32)])
    h = _linear(x, W, b)
    return _sc_aggregate(h, src, dst)


# speedup vs baseline: 1.2188x; 1.0007x over previous
import functools

import jax
import jax.numpy as jnp
from jax import lax
from jax.experimental import pallas as pl
from jax.experimental.pallas import tpu as pltpu
from jax.experimental.pallas import tpu_sc as plsc

N_NODES = 10000
D = 256
N_TILES = 32
ROWS = 312
ROWS_LAST = N_NODES - (N_TILES - 1) * ROWS
TRASH = 328
ACC_ROWS = 336
CHUNK_E = 4096
N_CHUNKS = 40
E_PAD = CHUNK_E * N_CHUNKS
BLK = 32
L = 16
UNROLL = 8


def _linear_body(x_ref, w_ref, b_ref, h_ref):
    h_ref[...] = (
        jnp.dot(x_ref[...], w_ref[...], preferred_element_type=jnp.float32)
        + b_ref[...]
    )


def _linear(x, W, b):
    return pl.pallas_call(
        _linear_body,
        grid=(10,),
        in_specs=[
            pl.BlockSpec((1000, D), lambda i: (i, 0)),
            pl.BlockSpec((D, D), lambda i: (0, 0)),
            pl.BlockSpec((1, D), lambda i: (0, 0)),
        ],
        out_specs=pl.BlockSpec((1000, D), lambda i: (i, 0)),
        out_shape=jax.ShapeDtypeStruct((N_NODES, D), jnp.float32),
    )(x, W, b.reshape(1, D))


def _sc_aggregate(h, src, dst):
    mesh = plsc.VectorSubcoreMesh(core_axis_name="c", subcore_axis_name="s")

    @functools.partial(
        pl.kernel,
        mesh=mesh,
        out_type=jax.ShapeDtypeStruct((N_NODES, D), jnp.float32),
        compiler_params=pltpu.CompilerParams(needs_layout_passes=False),
        scratch_types=[
            pltpu.VMEM((CHUNK_E,), jnp.int32),
            pltpu.VMEM((CHUNK_E,), jnp.int32),
            pltpu.VMEM((CHUNK_E,), jnp.int32),
            pltpu.VMEM((CHUNK_E,), jnp.int32),
            pltpu.VMEM((CHUNK_E + BLK,), jnp.int32),
            pltpu.VMEM((CHUNK_E + BLK,), jnp.int32),
            pltpu.VMEM((BLK, D), jnp.float32),
            pltpu.VMEM((BLK, D), jnp.float32),
            pltpu.VMEM((ACC_ROWS, D), jnp.float32),
            pltpu.SemaphoreType.DMA,
            pltpu.SemaphoreType.DMA,
            pltpu.SemaphoreType.DMA,
            pltpu.SemaphoreType.DMA,
        ],
    )
    def agg(h_hbm, src_hbm, dst_hbm, out_hbm,
            srcb0, dstb0, srcb1, dstb1, psrc, pldst, rows0, rows1, accum,
            sem0, sem1, stg0, stg1):
        cid = lax.axis_index("c")
        sid = lax.axis_index("s")
        wid = sid * 2 + cid
        row0 = wid * ROWS
        last = wid == N_TILES - 1
        nrows = jnp.where(last, ROWS_LAST, ROWS).astype(jnp.int32)
        lanes = lax.iota(jnp.int32, L)

        @pl.when(jnp.logical_not(last))
        def _():
            pltpu.sync_copy(h_hbm.at[pl.ds(row0, ROWS)], accum.at[pl.ds(0, ROWS)])

        @pl.when(last)
        def _():
            pltpu.sync_copy(
                h_hbm.at[pl.ds((N_TILES - 1) * ROWS, ROWS_LAST)],
                accum.at[pl.ds(0, ROWS_LAST)],
            )

        def stage_start(ch, sb, db, sgm):
            e0 = ch * CHUNK_E
            pltpu.make_async_copy(src_hbm.at[pl.ds(e0, CHUNK_E)], sb, sgm).start()
            pltpu.make_async_copy(dst_hbm.at[pl.ds(e0, CHUNK_E)], db, sgm).start()

        def stage_wait(sb, db, sgm):
            pltpu.make_async_copy(src_hbm.at[pl.ds(0, CHUNK_E)], sb, sgm).wait()
            pltpu.make_async_copy(dst_hbm.at[pl.ds(0, CHUNK_E)], db, sgm).wait()

        def gather_start(blkv, rows_b, sem_b):
            pltpu.make_async_copy(
                h_hbm.at[psrc.at[pl.ds(blkv * BLK, BLK)]], rows_b, sem_b
            ).start()

        def gather_wait(rows_b, sem_b):
            pltpu.make_async_copy(
                h_hbm.at[psrc.at[pl.ds(0, BLK)]], rows_b, sem_b
            ).wait()

        def add_block(blk, rows_b):
            def group_body(g, c2):
                ridx = pldst[pl.ds(blk * BLK + g * L, L)]
                for l in range(L):
                    r = jnp.sum(jnp.where(lanes == l, ridx, 0))
                    for cb in range(D // L):
                        vals = rows_b[g * L + l, pl.ds(cb * L, L)]
                        plsc.addupdate(accum.at[r, pl.ds(cb * L, L)], vals)
                return c2

            lax.fori_loop(0, BLK // L, group_body, 0)

        def chunk_work(srcb, dstb):
            def scan_group(g0, curv):
                for u in range(UNROLL):
                    g = g0 * UNROLL + u
                    d = dstb[pl.ds(g * L, L)]
                    s = srcb[pl.ds(g * L, L)]
                    adj = d - row0
                    m = (adj >= 0) & (adj < nrows)
                    mi = m.astype(jnp.int32)
                    inc = plsc.cumsum(mi)
                    pos = curv + inc - 1
                    plsc.store_scatter(pldst, [pos], adj, mask=m)
                    plsc.store_scatter(psrc, [pos], s, mask=m)
                    curv = curv + plsc.all_reduce_population_count(m)
                return curv

            curv = lax.fori_loop(
                0, CHUNK_E // L // UNROLL, scan_group,
                jnp.zeros((L,), jnp.int32),
            )
            cur = jnp.max(curv)

            nblk = (cur + BLK - 1) // BLK
            pad_end = nblk * BLK
            for t in range(BLK // L):
                off = t * L + lanes
                ppos = cur + off
                pm = ppos < pad_end
                plsc.store_scatter(
                    pldst, [ppos], jnp.full((L,), TRASH, jnp.int32), mask=pm
                )
                plsc.store_scatter(
                    psrc, [ppos], jnp.broadcast_to(row0, (L,)).astype(jnp.int32),
                    mask=pm,
                )

            @pl.when(nblk > 0)
            def _():
                gather_start(0, rows0, sem0)

            def pair_body(p, c2):
                b0 = 2 * p

                @pl.when(b0 + 1 < nblk)
                def _():
                    gather_start(b0 + 1, rows1, sem1)

                gather_wait(rows0, sem0)
                add_block(b0, rows0)

                @pl.when(b0 + 2 < nblk)
                def _():
                    gather_start(b0 + 2, rows0, sem0)

                @pl.when(b0 + 1 < nblk)
                def _():
                    gather_wait(rows1, sem1)
                    add_block(b0 + 1, rows1)

                return c2

            lax.fori_loop(0, (nblk + 1) // 2, pair_body, 0)

        stage_start(0, srcb0, dstb0, stg0)

        def chunk_pair(p, carry):
            ch0 = 2 * p
            stage_start(ch0 + 1, srcb1, dstb1, stg1)
            stage_wait(srcb0, dstb0, stg0)
            chunk_work(srcb0, dstb0)

            @pl.when(ch0 + 2 < N_CHUNKS)
            def _():
                stage_start(ch0 + 2, srcb0, dstb0, stg0)

            stage_wait(srcb1, dstb1, stg1)
            chunk_work(srcb1, dstb1)
            return carry

        lax.fori_loop(0, N_CHUNKS // 2, chunk_pair, 0)

        @pl.when(jnp.logical_not(last))
        def _():
            pltpu.sync_copy(
                accum.at[pl.ds(0, ROWS)], out_hbm.at[pl.ds(row0, ROWS)]
            )

        @pl.when(last)
        def _():
            pltpu.sync_copy(
                accum.at[pl.ds(0, ROWS_LAST)],
                out_hbm.at[pl.ds((N_TILES - 1) * ROWS, ROWS_LAST)],
            )

    return agg(h, src, dst)


def kernel(x, edge_index, W, b):
    e = edge_index.astype(jnp.int32)
    pad = E_PAD - e.shape[1]
    src = jnp.concatenate([e[0], jnp.zeros((pad,), jnp.int32)])
    dst = jnp.concatenate([e[1], jnp.full((pad,), -1, jnp.int32)])
    h = _linear(x, W, b)
    return _sc_aggregate(h, src, dst)
